# Initial kernel scaffold; baseline (speedup 1.0000x reference)
import jax
import jax.numpy as jnp
from jax.experimental import pallas as pl


def kernel(s, t, edges, edge_weight, sWl, sWr, sWe, satt, sbias, tWl, tWr, tWe, tatt, tbias):
    raise NotImplementedError("write your pallas kernel here")


# speedup vs baseline: 4.2154x; 4.2154x over previous
import functools

import jax
import jax.numpy as jnp
from jax import lax
from jax.experimental import pallas as pl
from jax.experimental.pallas import tpu as pltpu
from jax.experimental.pallas import tpu_sc as plsc

N = 10000
E = 160000
D = 256
H = 2
C = 128
NEG = 0.2
L = 16
NSUB = 16
NPAD = 10240
CHUNK = 80
NGRP = CHUNK // L
EPT = E // NSUB
NCH = EPT // CHUNK
RCH = N // CHUNK
ZCH = NPAD // CHUNK
SEG = NPAD // NSUB


def _mm_body(x_ref, w_ref, o_ref):
    o_ref[0] = jnp.dot(x_ref[...], w_ref[...],
                       preferred_element_type=jnp.float32)


def _project(x, w, bn):
    M, K = x.shape
    G = w.shape[1] // 128
    return pl.pallas_call(
        _mm_body,
        grid=(G, M // bn),
        in_specs=[
            pl.BlockSpec((bn, K), lambda g, i: (i, 0)),
            pl.BlockSpec((K, 128), lambda g, i: (0, g)),
        ],
        out_specs=pl.BlockSpec((1, bn, 128), lambda g, i: (g, i, 0)),
        out_shape=jax.ShapeDtypeStruct((G, M, 128), jnp.float32),
    )(x, w)


def _sc_body(e0, e1, xls, xrs, ees, xlt, xrt, eet, attb, biasb,
             outs, outt,
             xlbuf, xrbuf, eebuf, sidx, didx, didx2, exb, dz, den80,
             attv, biasv, out_sh, denf_sh, sem0, sem1, sem2):
    c = lax.axis_index("c")
    tid = lax.axis_index("s")
    iota = lax.iota(jnp.int32, L)
    zv = jnp.zeros((L,), jnp.float32)
    egrp = [iota + g * L for g in range(NGRP)]

    def _zdz(i, _):
        dz[pl.ds(i * L, L)] = zv
        return 0
    lax.fori_loop(0, SEG // L, _zdz, 0)

    for d, (xl, xr, ee, esrc, edst, outref) in enumerate((
            (xls, xrs, ees, e0, e1, outs),
            (xlt, xrt, eet, e1, e0, outt))):
        q = d * 2 + c
        pltpu.sync_copy(attb.at[pl.ds(q * C * L, C * L)], attv)
        pltpu.sync_copy(biasb.at[pl.ds(q * C * L, C * L)], biasv)

        def _zrow(r, _):
            for j in range(C // L):
                eebuf[r, pl.ds(j * L, L)] = zv
            return 0
        lax.fori_loop(0, CHUNK, _zrow, 0)

        def _zout(j, _):
            k = tid + j * NSUB
            pltpu.sync_copy(eebuf, out_sh.at[pl.ds(k * CHUNK, CHUNK)])
            return 0
        lax.fori_loop(0, ZCH // NSUB, _zout, 0)
        pltpu.sync_copy(dz, denf_sh.at[pl.ds(tid * SEG, SEG)])

        plsc.subcore_barrier()

        base0 = tid * EPT
        hoff = c * N

        def _echunk(k, _):
            base = base0 + k * CHUNK
            pltpu.sync_copy(esrc.at[pl.ds(base, CHUNK)], sidx)
            pltpu.sync_copy(edst.at[pl.ds(base, CHUNK)], didx)
            for g in range(NGRP):
                sl = pl.ds(g * L, L)
                sidx[sl] = sidx[sl] + hoff
                didx2[sl] = didx[sl] + hoff
            cp0 = pltpu.async_copy(xl.at[sidx], xlbuf, sem0)
            cp1 = pltpu.async_copy(xr.at[didx2], xrbuf, sem1)
            cp2 = pltpu.async_copy(ee.at[pl.ds(c * E + base, CHUNK)], eebuf,
                                   sem2)
            cp0.wait()
            cp1.wait()
            cp2.wait()

            def _alpha(cc, accs):
                attrow = attv[pl.ds(cc * L, L)]
                ccv = jnp.full((L,), cc, jnp.int32)
                out = []
                for g in range(NGRP):
                    xlv = plsc.load_gather(xlbuf, [egrp[g], ccv])
                    xrv = plsc.load_gather(xrbuf, [egrp[g], ccv])
                    eev = plsc.load_gather(eebuf, [egrp[g], ccv])
                    m = xlv + xrv + eev
                    m = jnp.maximum(m, NEG * m)
                    out.append(accs[g] + m * attrow)
                return tuple(out)
            accs = lax.fori_loop(0, C, _alpha, (zv,) * NGRP)
            exs = [jnp.exp(a) for a in accs]
            for g in range(NGRP):
                exb[pl.ds(g * L, L)] = exs[g]

            def _scale(cc, _):
                ccv = jnp.full((L,), cc, jnp.int32)
                for g in range(NGRP):
                    xlv = plsc.load_gather(xlbuf, [egrp[g], ccv])
                    plsc.store_scatter(xlbuf, [egrp[g], ccv], xlv * exs[g])
                return 0
            lax.fori_loop(0, C, _scale, 0)

            pltpu.sync_copy(exb, denf_sh.at[didx], add=True)
            pltpu.sync_copy(xlbuf, out_sh.at[didx], add=True)
            return 0
        lax.fori_loop(0, NCH, _echunk, 0)

        plsc.subcore_barrier()

        def _wchunk(j, _):
            k = tid + j * NSUB

            @pl.when(k < RCH)
            def _():
                rowbase = k * CHUNK
                pltpu.sync_copy(out_sh.at[pl.ds(rowbase, CHUNK)], xrbuf)
                pltpu.sync_copy(denf_sh.at[pl.ds(rowbase, CHUNK)], den80)
                rcps = [
                    1.0 / (den80[pl.ds(g * L, L)] + 1e-16)
                    for g in range(NGRP)
                ]

                def _nrm(cc, _):
                    biasrow = biasv[pl.ds(cc * L, L)]
                    ccv = jnp.full((L,), cc, jnp.int32)
                    for g in range(NGRP):
                        v = plsc.load_gather(xrbuf, [egrp[g], ccv])
                        v = jnp.maximum(v * rcps[g] + biasrow, 0.0)
                        plsc.store_scatter(xrbuf, [egrp[g], ccv], v)
                    return 0
                lax.fori_loop(0, C, _nrm, 0)
                pltpu.sync_copy(
                    xrbuf,
                    outref.at[pl.ds(rowbase, CHUNK), pl.ds(c * C, C)])
            return 0
        lax.fori_loop(0, (RCH + NSUB - 1) // NSUB, _wchunk, 0)
        plsc.subcore_barrier()


_sc_call = pl.kernel(
    _sc_body,
    out_type=(
        jax.ShapeDtypeStruct((N, H * C), jnp.float32),
        jax.ShapeDtypeStruct((N, H * C), jnp.float32),
    ),
    mesh=plsc.VectorSubcoreMesh(core_axis_name="c", subcore_axis_name="s"),
    compiler_params=pltpu.CompilerParams(needs_layout_passes=False),
    scratch_types=[
        pltpu.VMEM((CHUNK, C), jnp.float32),
        pltpu.VMEM((CHUNK, C), jnp.float32),
        pltpu.VMEM((CHUNK, C), jnp.float32),
        pltpu.VMEM((CHUNK,), jnp.int32),
        pltpu.VMEM((CHUNK,), jnp.int32),
        pltpu.VMEM((CHUNK,), jnp.int32),
        pltpu.VMEM((CHUNK,), jnp.float32),
        pltpu.VMEM((SEG,), jnp.float32),
        pltpu.VMEM((CHUNK,), jnp.float32),
        pltpu.VMEM((C * L,), jnp.float32),
        pltpu.VMEM((C * L,), jnp.float32),
        pltpu.VMEM_SHARED((NPAD, C), jnp.float32),
        pltpu.VMEM_SHARED((NPAD,), jnp.float32),
        pltpu.SemaphoreType.DMA,
        pltpu.SemaphoreType.DMA,
        pltpu.SemaphoreType.DMA,
    ],
)


@jax.jit
def kernel(s, t, edges, edge_weight,
           sWl, sWr, sWe, satt, sbias,
           tWl, tWr, tWe, tatt, tbias):
    pa = _project(s, jnp.concatenate([sWl, tWr], axis=1), 1000)
    pb = _project(t, jnp.concatenate([sWr, tWl], axis=1), 1000)
    pc = _project(edge_weight, jnp.concatenate([sWe, tWe], axis=1), 2000)
    xls = pa[0:2].reshape(2 * N, C)
    xrt = pa[2:4].reshape(2 * N, C)
    xrs = pb[0:2].reshape(2 * N, C)
    xlt = pb[2:4].reshape(2 * N, C)
    ees = pc[0:2].reshape(2 * E, C)
    eet = pc[2:4].reshape(2 * E, C)

    att4 = jnp.concatenate([satt, tatt], axis=0)
    attb = jnp.broadcast_to(att4[:, :, None], (4, C, L)).reshape(4 * C * L)
    bias4 = jnp.concatenate(
        [sbias.reshape(H, C), tbias.reshape(H, C)], axis=0)
    biasb = jnp.broadcast_to(bias4[:, :, None], (4, C, L)).reshape(4 * C * L)

    outs, outt = _sc_call(edges[0], edges[1], xls, xrs, ees, xlt, xrt, eet,
                          attb, biasb)
    return (outs, outt, edges, edge_weight)


# speedup vs baseline: 5.0856x; 1.2065x over previous
import functools

import jax
import jax.numpy as jnp
from jax import lax
from jax.experimental import pallas as pl
from jax.experimental.pallas import tpu as pltpu
from jax.experimental.pallas import tpu_sc as plsc

N = 10000
E = 160000
D = 256
H = 2
C = 128
NEG = 0.2
L = 16
NSUB = 16
NPAD = 10240
CHUNK = 80
NGRP = CHUNK // L
EPT = E // NSUB
NCH = EPT // CHUNK
RCH = N // CHUNK
ZCH = NPAD // CHUNK
SEG = NPAD // NSUB


def _mm_body(x_ref, w_ref, o_ref):
    o_ref[0] = jnp.dot(x_ref[...], w_ref[...],
                       preferred_element_type=jnp.float32)


def _project(x, w, bn):
    M, K = x.shape
    G = w.shape[1] // 128
    return pl.pallas_call(
        _mm_body,
        grid=(G, M // bn),
        in_specs=[
            pl.BlockSpec((bn, K), lambda g, i: (i, 0)),
            pl.BlockSpec((K, 128), lambda g, i: (0, g)),
        ],
        out_specs=pl.BlockSpec((1, bn, 128), lambda g, i: (g, i, 0)),
        out_shape=jax.ShapeDtypeStruct((G, M, 128), jnp.float32),
    )(x, w)


def _sc_body(e0, e1, xls, xrs, ees, xlt, xrt, eet, attb, biasb,
             outs, outt,
             xlbuf, xrbuf, eebuf, sidx, didx, didx2, exb, dz, den80,
             attv, biasv, out_sh, denf_sh, sem0, sem1, sem2):
    c = lax.axis_index("c")
    tid = lax.axis_index("s")
    iota = lax.iota(jnp.int32, L)
    zv = jnp.zeros((L,), jnp.float32)
    egrp = [iota + g * L for g in range(NGRP)]

    def _zdz(i, _):
        dz[pl.ds(i * L, L)] = zv
        return 0
    lax.fori_loop(0, SEG // L, _zdz, 0)

    for d, (xl, xr, ee, esrc, edst, outref) in enumerate((
            (xls, xrs, ees, e0, e1, outs),
            (xlt, xrt, eet, e1, e0, outt))):
        q = d * 2 + c
        pltpu.sync_copy(attb.at[pl.ds(q * C * L, C * L)], attv)
        pltpu.sync_copy(biasb.at[pl.ds(q * C * L, C * L)], biasv)

        def _zrow(r, _):
            for j in range(C // L):
                eebuf[r, pl.ds(j * L, L)] = zv
            return 0
        lax.fori_loop(0, CHUNK, _zrow, 0)

        def _zout(j, _):
            k = tid + j * NSUB
            pltpu.sync_copy(eebuf, out_sh.at[pl.ds(k * CHUNK, CHUNK)])
            return 0
        lax.fori_loop(0, ZCH // NSUB, _zout, 0)
        pltpu.sync_copy(dz, denf_sh.at[pl.ds(tid * SEG, SEG)])

        plsc.subcore_barrier()

        base0 = tid * EPT
        hoff = c * N

        def _echunk(k, _):
            base = base0 + k * CHUNK
            pltpu.sync_copy(esrc.at[pl.ds(base, CHUNK)], sidx)
            pltpu.sync_copy(edst.at[pl.ds(base, CHUNK)], didx)
            for g in range(NGRP):
                sl = pl.ds(g * L, L)
                sidx[sl] = sidx[sl] + hoff
                didx2[sl] = didx[sl] + hoff
            cp0 = pltpu.async_copy(xl.at[sidx], xlbuf, sem0)
            cp1 = pltpu.async_copy(xr.at[didx2], xrbuf, sem1)
            cp2 = pltpu.async_copy(ee.at[pl.ds(c * E + base, CHUNK)], eebuf,
                                   sem2)
            cp0.wait()
            cp1.wait()
            cp2.wait()

            @plsc.parallel_loop(0, C, unroll=8, carry=(zv,) * NGRP)
            def _alpha(cc, accs):
                attrow = attv[pl.ds(cc * L, L)]
                ccv = jnp.full((L,), cc, jnp.int32)
                out = []
                for g in range(NGRP):
                    xlv = plsc.load_gather(xlbuf, [egrp[g], ccv])
                    xrv = plsc.load_gather(xrbuf, [egrp[g], ccv])
                    eev = plsc.load_gather(eebuf, [egrp[g], ccv])
                    m = xlv + xrv + eev
                    m = jnp.maximum(m, NEG * m)
                    out.append(accs[g] + m * attrow)
                return tuple(out)
            accs = _alpha
            exs = [jnp.exp(a) for a in accs]
            for g in range(NGRP):
                exb[pl.ds(g * L, L)] = exs[g]

            @plsc.parallel_loop(0, C, unroll=8)
            def _scale(cc):
                ccv = jnp.full((L,), cc, jnp.int32)
                for g in range(NGRP):
                    xlv = plsc.load_gather(xlbuf, [egrp[g], ccv])
                    plsc.store_scatter(xlbuf, [egrp[g], ccv], xlv * exs[g])

            pltpu.sync_copy(exb, denf_sh.at[didx], add=True)
            pltpu.sync_copy(xlbuf, out_sh.at[didx], add=True)
            return 0
        lax.fori_loop(0, NCH, _echunk, 0)

        plsc.subcore_barrier()

        def _wchunk(j, _):
            k = tid + j * NSUB

            @pl.when(k < RCH)
            def _():
                rowbase = k * CHUNK
                pltpu.sync_copy(out_sh.at[pl.ds(rowbase, CHUNK)], xrbuf)
                pltpu.sync_copy(denf_sh.at[pl.ds(rowbase, CHUNK)], den80)
                rcps = [
                    1.0 / (den80[pl.ds(g * L, L)] + 1e-16)
                    for g in range(NGRP)
                ]

                @plsc.parallel_loop(0, C, unroll=8)
                def _nrm(cc):
                    biasrow = biasv[pl.ds(cc * L, L)]
                    ccv = jnp.full((L,), cc, jnp.int32)
                    for g in range(NGRP):
                        v = plsc.load_gather(xrbuf, [egrp[g], ccv])
                        v = jnp.maximum(v * rcps[g] + biasrow, 0.0)
                        plsc.store_scatter(xrbuf, [egrp[g], ccv], v)
                pltpu.sync_copy(
                    xrbuf,
                    outref.at[pl.ds(rowbase, CHUNK), pl.ds(c * C, C)])
            return 0
        lax.fori_loop(0, (RCH + NSUB - 1) // NSUB, _wchunk, 0)
        plsc.subcore_barrier()


_sc_call = pl.kernel(
    _sc_body,
    out_type=(
        jax.ShapeDtypeStruct((N, H * C), jnp.float32),
        jax.ShapeDtypeStruct((N, H * C), jnp.float32),
    ),
    mesh=plsc.VectorSubcoreMesh(core_axis_name="c", subcore_axis_name="s"),
    compiler_params=pltpu.CompilerParams(needs_layout_passes=False),
    scratch_types=[
        pltpu.VMEM((CHUNK, C), jnp.float32),
        pltpu.VMEM((CHUNK, C), jnp.float32),
        pltpu.VMEM((CHUNK, C), jnp.float32),
        pltpu.VMEM((CHUNK,), jnp.int32),
        pltpu.VMEM((CHUNK,), jnp.int32),
        pltpu.VMEM((CHUNK,), jnp.int32),
        pltpu.VMEM((CHUNK,), jnp.float32),
        pltpu.VMEM((SEG,), jnp.float32),
        pltpu.VMEM((CHUNK,), jnp.float32),
        pltpu.VMEM((C * L,), jnp.float32),
        pltpu.VMEM((C * L,), jnp.float32),
        pltpu.VMEM_SHARED((NPAD, C), jnp.float32),
        pltpu.VMEM_SHARED((NPAD,), jnp.float32),
        pltpu.SemaphoreType.DMA,
        pltpu.SemaphoreType.DMA,
        pltpu.SemaphoreType.DMA,
    ],
)


@jax.jit
def kernel(s, t, edges, edge_weight,
           sWl, sWr, sWe, satt, sbias,
           tWl, tWr, tWe, tatt, tbias):
    pa = _project(s, jnp.concatenate([sWl, tWr], axis=1), 1000)
    pb = _project(t, jnp.concatenate([sWr, tWl], axis=1), 1000)
    pc = _project(edge_weight, jnp.concatenate([sWe, tWe], axis=1), 2000)
    xls = pa[0:2].reshape(2 * N, C)
    xrt = pa[2:4].reshape(2 * N, C)
    xrs = pb[0:2].reshape(2 * N, C)
    xlt = pb[2:4].reshape(2 * N, C)
    ees = pc[0:2].reshape(2 * E, C)
    eet = pc[2:4].reshape(2 * E, C)

    att4 = jnp.concatenate([satt, tatt], axis=0)
    attb = jnp.broadcast_to(att4[:, :, None], (4, C, L)).reshape(4 * C * L)
    bias4 = jnp.concatenate(
        [sbias.reshape(H, C), tbias.reshape(H, C)], axis=0)
    biasb = jnp.broadcast_to(bias4[:, :, None], (4, C, L)).reshape(4 * C * L)

    outs, outt = _sc_call(edges[0], edges[1], xls, xrs, ees, xlt, xrt, eet,
                          attb, biasb)
    return (outs, outt, edges, edge_weight)


# speedup vs baseline: 19.4772x; 3.8298x over previous
import functools

import jax
import jax.numpy as jnp
from jax import lax
from jax.experimental import pallas as pl
from jax.experimental.pallas import tpu as pltpu
from jax.experimental.pallas import tpu_sc as plsc

N = 10000
E = 160000
D = 256
H = 2
C = 128
NEG = 0.2
L = 16
NSUB = 16
NPAD = 10240
CHUNK = 80
NGRP = CHUNK // L
EPT = E // NSUB
NCH = EPT // CHUNK
RCH = N // CHUNK
ZCH = NPAD // CHUNK
SEG = NPAD // NSUB


def _mm_body(x_ref, w_ref, o_ref):
    o_ref[0] = jnp.dot(x_ref[...], w_ref[...],
                       preferred_element_type=jnp.float32)


def _project(x, w, bn):
    M, K = x.shape
    G = w.shape[1] // 128
    return pl.pallas_call(
        _mm_body,
        grid=(G, M // bn),
        in_specs=[
            pl.BlockSpec((bn, K), lambda g, i: (i, 0)),
            pl.BlockSpec((K, 128), lambda g, i: (0, g)),
        ],
        out_specs=pl.BlockSpec((1, bn, 128), lambda g, i: (g, i, 0)),
        out_shape=jax.ShapeDtypeStruct((G, M, 128), jnp.float32),
    )(x, w)


def _sc_body(e0, e1, xls, xrs, ees, xlt, xrt, eet, attb, biasb,
             outs, outt,
             xlbuf, xrbuf, eebuf, sidx, didx, didx2, exb, dz, den80,
             attv, biasv, out_sh, denf_sh, sem0, sem1, sem2):
    c = lax.axis_index("c")
    tid = lax.axis_index("s")
    iota = lax.iota(jnp.int32, L)
    zv = jnp.zeros((L,), jnp.float32)
    egrp = [iota + g * L for g in range(NGRP)]

    def _zdz(i, _):
        dz[pl.ds(i * L, L)] = zv
        return 0
    lax.fori_loop(0, SEG // L, _zdz, 0)

    for d, (xl, xr, ee, esrc, edst, outref) in enumerate((
            (xls, xrs, ees, e0, e1, outs),
            (xlt, xrt, eet, e1, e0, outt))):
        q = d * 2 + c
        pltpu.sync_copy(attb.at[pl.ds(q * C, C)], attv)
        pltpu.sync_copy(biasb.at[pl.ds(q * C, C)], biasv)

        def _zrow(r, _):
            for j in range(C // L):
                eebuf[r, pl.ds(j * L, L)] = zv
            return 0
        lax.fori_loop(0, CHUNK, _zrow, 0)

        def _zout(j, _):
            k = tid + j * NSUB
            pltpu.sync_copy(eebuf, out_sh.at[pl.ds(k * CHUNK, CHUNK)])
            return 0
        lax.fori_loop(0, ZCH // NSUB, _zout, 0)
        pltpu.sync_copy(dz, denf_sh.at[pl.ds(tid * SEG, SEG)])

        plsc.subcore_barrier()

        base0 = tid * EPT
        hoff = c * N

        def _echunk(k, _):
            base = base0 + k * CHUNK
            pltpu.sync_copy(esrc.at[pl.ds(base, CHUNK)], sidx)
            pltpu.sync_copy(edst.at[pl.ds(base, CHUNK)], didx)
            for g in range(NGRP):
                sl = pl.ds(g * L, L)
                sidx[sl] = sidx[sl] + hoff
                didx2[sl] = didx[sl] + hoff
            cp0 = pltpu.async_copy(xl.at[sidx], xlbuf, sem0)
            cp1 = pltpu.async_copy(xr.at[didx2], xrbuf, sem1)
            cp2 = pltpu.async_copy(ee.at[pl.ds(c * E + base, CHUNK)], eebuf,
                                   sem2)
            cp0.wait()
            cp1.wait()
            cp2.wait()

            lane0 = iota == 0

            @plsc.parallel_loop(0, CHUNK, unroll=4)
            def _alpha(e):
                acc = zv
                for j in range(C // L):
                    sl = pl.ds(j * L, L)
                    m = xlbuf[e, sl] + xrbuf[e, sl] + eebuf[e, sl]
                    m = jnp.maximum(m, NEG * m)
                    acc = acc + m * attv[sl]
                ex = jnp.exp(jnp.full((L,), jnp.sum(acc), jnp.float32))
                plsc.store_scatter(exb, [jnp.full((L,), e, jnp.int32)], ex,
                                   mask=lane0)

            @plsc.parallel_loop(0, CHUNK, unroll=4)
            def _scale(e):
                exv = plsc.load_gather(exb, [jnp.full((L,), e, jnp.int32)])
                for j in range(C // L):
                    sl = pl.ds(j * L, L)
                    xlbuf[e, sl] = xlbuf[e, sl] * exv

            pltpu.sync_copy(exb, denf_sh.at[didx], add=True)
            pltpu.sync_copy(xlbuf, out_sh.at[didx], add=True)
            return 0
        lax.fori_loop(0, NCH, _echunk, 0)

        plsc.subcore_barrier()

        def _wchunk(j, _):
            k = tid + j * NSUB

            @pl.when(k < RCH)
            def _():
                rowbase = k * CHUNK
                pltpu.sync_copy(out_sh.at[pl.ds(rowbase, CHUNK)], xrbuf)
                pltpu.sync_copy(denf_sh.at[pl.ds(rowbase, CHUNK)], den80)
                @plsc.parallel_loop(0, CHUNK, unroll=4)
                def _nrm(r):
                    dv = plsc.load_gather(
                        den80, [jnp.full((L,), r, jnp.int32)])
                    rcv = 1.0 / (dv + 1e-16)
                    for j in range(C // L):
                        sl = pl.ds(j * L, L)
                        v = xrbuf[r, sl] * rcv + biasv[sl]
                        xrbuf[r, sl] = jnp.maximum(v, 0.0)
                pltpu.sync_copy(
                    xrbuf,
                    outref.at[pl.ds(rowbase, CHUNK), pl.ds(c * C, C)])
            return 0
        lax.fori_loop(0, (RCH + NSUB - 1) // NSUB, _wchunk, 0)
        plsc.subcore_barrier()


_sc_call = pl.kernel(
    _sc_body,
    out_type=(
        jax.ShapeDtypeStruct((N, H * C), jnp.float32),
        jax.ShapeDtypeStruct((N, H * C), jnp.float32),
    ),
    mesh=plsc.VectorSubcoreMesh(core_axis_name="c", subcore_axis_name="s"),
    compiler_params=pltpu.CompilerParams(needs_layout_passes=False),
    scratch_types=[
        pltpu.VMEM((CHUNK, C), jnp.float32),
        pltpu.VMEM((CHUNK, C), jnp.float32),
        pltpu.VMEM((CHUNK, C), jnp.float32),
        pltpu.VMEM((CHUNK,), jnp.int32),
        pltpu.VMEM((CHUNK,), jnp.int32),
        pltpu.VMEM((CHUNK,), jnp.int32),
        pltpu.VMEM((CHUNK,), jnp.float32),
        pltpu.VMEM((SEG,), jnp.float32),
        pltpu.VMEM((CHUNK,), jnp.float32),
        pltpu.VMEM((C,), jnp.float32),
        pltpu.VMEM((C,), jnp.float32),
        pltpu.VMEM_SHARED((NPAD, C), jnp.float32),
        pltpu.VMEM_SHARED((NPAD,), jnp.float32),
        pltpu.SemaphoreType.DMA,
        pltpu.SemaphoreType.DMA,
        pltpu.SemaphoreType.DMA,
    ],
)


@jax.jit
def kernel(s, t, edges, edge_weight,
           sWl, sWr, sWe, satt, sbias,
           tWl, tWr, tWe, tatt, tbias):
    pa = _project(s, jnp.concatenate([sWl, tWr], axis=1), 1000)
    pb = _project(t, jnp.concatenate([sWr, tWl], axis=1), 1000)
    pc = _project(edge_weight, jnp.concatenate([sWe, tWe], axis=1), 2000)
    xls = pa[0:2].reshape(2 * N, C)
    xrt = pa[2:4].reshape(2 * N, C)
    xrs = pb[0:2].reshape(2 * N, C)
    xlt = pb[2:4].reshape(2 * N, C)
    ees = pc[0:2].reshape(2 * E, C)
    eet = pc[2:4].reshape(2 * E, C)

    att4 = jnp.concatenate([satt, tatt], axis=0)
    attb = att4.reshape(4 * C)
    bias4 = jnp.concatenate(
        [sbias.reshape(H, C), tbias.reshape(H, C)], axis=0)
    biasb = bias4.reshape(4 * C)

    outs, outt = _sc_call(edges[0], edges[1], xls, xrs, ees, xlt, xrt, eet,
                          attb, biasb)
    return (outs, outt, edges, edge_weight)


# speedup vs baseline: 23.5249x; 1.2078x over previous
import functools

import jax
import jax.numpy as jnp
from jax import lax
from jax.experimental import pallas as pl
from jax.experimental.pallas import tpu as pltpu
from jax.experimental.pallas import tpu_sc as plsc

N = 10000
E = 160000
D = 256
H = 2
C = 128
NEG = 0.2
L = 16
NSUB = 16
CHUNK = 64
NGRP = CHUNK // L
NCHG = E // CHUNK
JPAD = (NCHG + NSUB - 1) // NSUB
NPAD = JPAD * CHUNK
WFULL = N // CHUNK
WTAIL = N - WFULL * CHUNK


def _mm_body(x_ref, w_ref, o_ref):
    o_ref[0] = jnp.dot(x_ref[...], w_ref[...],
                       preferred_element_type=jnp.float32)


def _project(x, w, bn):
    M, K = x.shape
    G = w.shape[1] // 128
    return pl.pallas_call(
        _mm_body,
        grid=(G, M // bn),
        in_specs=[
            pl.BlockSpec((bn, K), lambda g, i: (i, 0)),
            pl.BlockSpec((K, 128), lambda g, i: (0, g)),
        ],
        out_specs=pl.BlockSpec((1, bn, 128), lambda g, i: (g, i, 0)),
        out_shape=jax.ShapeDtypeStruct((G, M, 128), jnp.float32),
    )(x, w)


def _sc_body(e0, e1, xls, xrs, ees, xlt, xrt, eet, attb, biasb,
             outs, outt,
             xlb0, xlb1, xrb0, xrb1, eeb0, eeb1, eib0, eib1,
             exb, attv, biasv,
             out_sh, denf_sh,
             semxl0, semxl1, semxr0, semxr1, semee0, semee1):
    c = lax.axis_index("c")
    tid = lax.axis_index("s")
    iota = lax.iota(jnp.int32, L)
    zv = jnp.zeros((L,), jnp.float32)
    lane0 = iota == 0
    xlb = (xlb0, xlb1)
    xrb = (xrb0, xrb1)
    eeb = (eeb0, eeb1)
    eib = (eib0, eib1)
    semxl = (semxl0, semxl1)
    semxr = (semxr0, semxr1)
    semee = (semee0, semee1)

    for d, (xl, xr, ee, esrc, edst, outref) in enumerate((
            (xls, xrs, ees, e0, e1, outs),
            (xlt, xrt, eet, e1, e0, outt))):
        q = d * 2 + c
        pltpu.sync_copy(attb.at[pl.ds(q * C, C)], attv)
        pltpu.sync_copy(biasb.at[pl.ds(q * C, C)], biasv)
        hoff = c * N

        def _zrow(r, _):
            for jz in range(C // L):
                xlb0[r, pl.ds(jz * L, L)] = zv
            return 0
        lax.fori_loop(0, CHUNK, _zrow, 0)
        for g in range(NGRP):
            exb[pl.ds(g * L, L)] = zv

        def _zout(jz, _):
            k = tid + jz * NSUB

            @pl.when(k < WFULL)
            def _():
                pltpu.sync_copy(xlb0, out_sh.at[pl.ds(k * CHUNK, CHUNK)])

            @pl.when(k == WFULL)
            def _():
                pltpu.sync_copy(xlb0.at[pl.ds(0, WTAIL)],
                                out_sh.at[pl.ds(WFULL * CHUNK, WTAIL)])

            @pl.when(k < JPAD)
            def _():
                pltpu.sync_copy(exb, denf_sh.at[pl.ds(k * CHUNK, CHUNK)])
            return 0
        lax.fori_loop(0, (JPAD + NSUB - 1) // NSUB, _zout, 0)

        plsc.subcore_barrier()

        def _stage(j, b):
            k = tid + j * NSUB
            base = jnp.minimum(k, NCHG - 1) * CHUNK
            pltpu.sync_copy(esrc.at[pl.ds(base, CHUNK)], eib[b].at[0])
            pltpu.sync_copy(edst.at[pl.ds(base, CHUNK)], eib[b].at[1])
            for g in range(NGRP):
                sl = pl.ds(g * L, L)
                eib[b][0, sl] = eib[b][0, sl] + hoff
                eib[b][1, sl] = eib[b][1, sl] + hoff
            pltpu.async_copy(xl.at[eib[b].at[0]], xlb[b], semxl[b])
            pltpu.async_copy(xr.at[eib[b].at[1]], xrb[b], semxr[b])
            pltpu.async_copy(ee.at[pl.ds(c * E + base, CHUNK)], eeb[b],
                             semee[b])

        def _compute(j, b):
            valid = (tid + j * NSUB) < NCHG
            vs = jnp.full((L,), jnp.where(valid, 1.0, 0.0), jnp.float32)
            mxl = xlb[b]
            mxr = xrb[b]
            mee = eeb[b]

            @plsc.parallel_loop(0, CHUNK, unroll=4)
            def _alpha(e):
                acc = zv
                for jj in range(C // L):
                    sl = pl.ds(jj * L, L)
                    m = mxl[e, sl] + mxr[e, sl] + mee[e, sl]
                    m = jnp.maximum(m, NEG * m)
                    acc = acc + m * attv[sl]
                ex = jnp.exp(jnp.full((L,), jnp.sum(acc), jnp.float32)) * vs
                plsc.store_scatter(exb, [jnp.full((L,), e, jnp.int32)], ex,
                                   mask=lane0)

            @plsc.parallel_loop(0, CHUNK, unroll=4)
            def _scale(e):
                exv = plsc.load_gather(exb, [jnp.full((L,), e, jnp.int32)])
                for jj in range(C // L):
                    sl = pl.ds(jj * L, L)
                    mxl[e, sl] = mxl[e, sl] * exv

            for g in range(NGRP):
                sl = pl.ds(g * L, L)
                eib[b][1, sl] = eib[b][1, sl] - hoff
            pltpu.sync_copy(exb, denf_sh.at[eib[b].at[1]], add=True)
            pltpu.sync_copy(mxl, out_sh.at[eib[b].at[1]], add=True)

        _stage(0, 0)

        def _pair(jo, _):
            for b in range(2):
                j = jo * 2 + b

                @pl.when(j < JPAD)
                def _():
                    k = tid + j * NSUB
                    base = jnp.minimum(k, NCHG - 1) * CHUNK
                    pltpu.make_async_copy(xl.at[eib[b].at[0]], xlb[b],
                                          semxl[b]).wait()
                    pltpu.make_async_copy(xr.at[eib[b].at[1]], xrb[b],
                                          semxr[b]).wait()
                    pltpu.make_async_copy(
                        ee.at[pl.ds(c * E + base, CHUNK)], eeb[b],
                        semee[b]).wait()

                    @pl.when(j + 1 < JPAD)
                    def _():
                        _stage(j + 1, 1 - b)
                    _compute(j, b)
            return 0
        lax.fori_loop(0, (JPAD + 1) // 2, _pair, 0)

        plsc.subcore_barrier()

        def _wb(rows, rowbase):
            pltpu.sync_copy(out_sh.at[pl.ds(rowbase, rows)],
                            xrb0.at[pl.ds(0, rows)])
            pltpu.sync_copy(denf_sh.at[pl.ds(rowbase, rows)],
                            exb.at[pl.ds(0, rows)])

            @plsc.parallel_loop(0, rows, unroll=4)
            def _nrm(r):
                dv = plsc.load_gather(exb, [jnp.full((L,), r, jnp.int32)])
                rcv = 1.0 / (dv + 1e-16)
                for jj in range(C // L):
                    sl = pl.ds(jj * L, L)
                    v = xrb0[r, sl] * rcv + biasv[sl]
                    xrb0[r, sl] = jnp.maximum(v, 0.0)
            pltpu.sync_copy(
                xrb0.at[pl.ds(0, rows)],
                outref.at[pl.ds(rowbase, rows), pl.ds(c * C, C)])

        def _wchunk(jw, _):
            k = tid + jw * NSUB

            @pl.when(k < WFULL)
            def _():
                _wb(CHUNK, k * CHUNK)

            @pl.when(k == WFULL)
            def _():
                _wb(WTAIL, WFULL * CHUNK)
            return 0
        lax.fori_loop(0, (WFULL + NSUB) // NSUB, _wchunk, 0)
        plsc.subcore_barrier()


_sc_call = pl.kernel(
    _sc_body,
    out_type=(
        jax.ShapeDtypeStruct((N, H * C), jnp.float32),
        jax.ShapeDtypeStruct((N, H * C), jnp.float32),
    ),
    mesh=plsc.VectorSubcoreMesh(core_axis_name="c", subcore_axis_name="s"),
    compiler_params=pltpu.CompilerParams(needs_layout_passes=False),
    scratch_types=[
        pltpu.VMEM((CHUNK, C), jnp.float32),
        pltpu.VMEM((CHUNK, C), jnp.float32),
        pltpu.VMEM((CHUNK, C), jnp.float32),
        pltpu.VMEM((CHUNK, C), jnp.float32),
        pltpu.VMEM((CHUNK, C), jnp.float32),
        pltpu.VMEM((CHUNK, C), jnp.float32),
        pltpu.VMEM((2, CHUNK), jnp.int32),
        pltpu.VMEM((2, CHUNK), jnp.int32),
        pltpu.VMEM((CHUNK,), jnp.float32),
        pltpu.VMEM((C,), jnp.float32),
        pltpu.VMEM((C,), jnp.float32),
        pltpu.VMEM_SHARED((N, C), jnp.float32),
        pltpu.VMEM_SHARED((NPAD,), jnp.float32),
        pltpu.SemaphoreType.DMA,
        pltpu.SemaphoreType.DMA,
        pltpu.SemaphoreType.DMA,
        pltpu.SemaphoreType.DMA,
        pltpu.SemaphoreType.DMA,
        pltpu.SemaphoreType.DMA,
    ],
)


@jax.jit
def kernel(s, t, edges, edge_weight,
           sWl, sWr, sWe, satt, sbias,
           tWl, tWr, tWe, tatt, tbias):
    pa = _project(s, jnp.concatenate([sWl, tWr], axis=1), 1000)
    pb = _project(t, jnp.concatenate([sWr, tWl], axis=1), 1000)
    pc = _project(edge_weight, jnp.concatenate([sWe, tWe], axis=1), 2000)
    xls = pa[0:2].reshape(2 * N, C)
    xrt = pa[2:4].reshape(2 * N, C)
    xrs = pb[0:2].reshape(2 * N, C)
    xlt = pb[2:4].reshape(2 * N, C)
    ees = pc[0:2].reshape(2 * E, C)
    eet = pc[2:4].reshape(2 * E, C)

    att4 = jnp.concatenate([satt, tatt], axis=0)
    attb = att4.reshape(4 * C)
    bias4 = jnp.concatenate(
        [sbias.reshape(H, C), tbias.reshape(H, C)], axis=0)
    biasb = bias4.reshape(4 * C)

    outs, outt = _sc_call(edges[0], edges[1], xls, xrs, ees, xlt, xrt,
                          eet, attb, biasb)
    return (outs, outt, edges, edge_weight)


# speedup vs baseline: 23.8031x; 1.0118x over previous
import functools

import jax
import jax.numpy as jnp
from jax import lax
from jax.experimental import pallas as pl
from jax.experimental.pallas import tpu as pltpu
from jax.experimental.pallas import tpu_sc as plsc

N = 10000
E = 160000
D = 256
H = 2
C = 128
NEG = 0.2
L = 16
NSUB = 16
CHUNK = 64
NGRP = CHUNK // L
NCHG = E // CHUNK
JPAD = (NCHG + NSUB - 1) // NSUB
NPAD = JPAD * CHUNK
WFULL = N // CHUNK
WTAIL = N - WFULL * CHUNK


def _mm_body(x_ref, w_ref, o_ref):
    o_ref[0] = jnp.dot(x_ref[...], w_ref[...],
                       preferred_element_type=jnp.float32)


def _project(x, w, bn):
    M, K = x.shape
    G = w.shape[1] // 128
    return pl.pallas_call(
        _mm_body,
        grid=(G, M // bn),
        in_specs=[
            pl.BlockSpec((bn, K), lambda g, i: (i, 0)),
            pl.BlockSpec((K, 128), lambda g, i: (0, g)),
        ],
        out_specs=pl.BlockSpec((1, bn, 128), lambda g, i: (g, i, 0)),
        out_shape=jax.ShapeDtypeStruct((G, M, 128), jnp.float32),
    )(x, w)


def _sc_body(e0, e1, xls, xrs, ees, xlt, xrt, eet, attb, biasb,
             outs, outt,
             xlb0, xlb1, xrb0, xrb1, eeb0, eeb1, eib0, eib1,
             exb0, exb1, attv, biasv,
             out_sh, denf_sh,
             semxl0, semxl1, semxr0, semxr1, semee0, semee1,
             semso0, semso1, semsd0, semsd1):
    c = lax.axis_index("c")
    tid = lax.axis_index("s")
    iota = lax.iota(jnp.int32, L)
    zv = jnp.zeros((L,), jnp.float32)
    lane0 = iota == 0
    xlb = (xlb0, xlb1)
    xrb = (xrb0, xrb1)
    eeb = (eeb0, eeb1)
    eib = (eib0, eib1)
    semxl = (semxl0, semxl1)
    semxr = (semxr0, semxr1)
    semee = (semee0, semee1)
    semso = (semso0, semso1)
    semsd = (semsd0, semsd1)
    exb = (exb0, exb1)

    for d, (xl, xr, ee, esrc, edst, outref) in enumerate((
            (xls, xrs, ees, e0, e1, outs),
            (xlt, xrt, eet, e1, e0, outt))):
        q = d * 2 + c
        pltpu.sync_copy(attb.at[pl.ds(q * C, C)], attv)
        pltpu.sync_copy(biasb.at[pl.ds(q * C, C)], biasv)
        hoff = c * N

        def _zrow(r, _):
            for jz in range(C // L):
                xlb0[r, pl.ds(jz * L, L)] = zv
            return 0
        lax.fori_loop(0, CHUNK, _zrow, 0)
        for g in range(NGRP):
            exb0[pl.ds(g * L, L)] = zv

        def _zout(jz, _):
            k = tid + jz * NSUB

            @pl.when(k < WFULL)
            def _():
                pltpu.sync_copy(xlb0, out_sh.at[pl.ds(k * CHUNK, CHUNK)])

            @pl.when(k == WFULL)
            def _():
                pltpu.sync_copy(xlb0.at[pl.ds(0, WTAIL)],
                                out_sh.at[pl.ds(WFULL * CHUNK, WTAIL)])

            @pl.when(k < JPAD)
            def _():
                pltpu.sync_copy(exb0, denf_sh.at[pl.ds(k * CHUNK, CHUNK)])
            return 0
        lax.fori_loop(0, (JPAD + NSUB - 1) // NSUB, _zout, 0)

        plsc.subcore_barrier()

        def _stage(j, b):
            k = tid + j * NSUB
            base = jnp.minimum(k, NCHG - 1) * CHUNK
            pltpu.sync_copy(esrc.at[pl.ds(base, CHUNK)], eib[b].at[0])
            pltpu.sync_copy(edst.at[pl.ds(base, CHUNK)], eib[b].at[1])
            for g in range(NGRP):
                sl = pl.ds(g * L, L)
                eib[b][0, sl] = eib[b][0, sl] + hoff
                eib[b][1, sl] = eib[b][1, sl] + hoff
            pltpu.async_copy(xl.at[eib[b].at[0]], xlb[b], semxl[b])
            pltpu.async_copy(xr.at[eib[b].at[1]], xrb[b], semxr[b])
            pltpu.async_copy(ee.at[pl.ds(c * E + base, CHUNK)], eeb[b],
                             semee[b])

        def _compute(j, b):
            valid = (tid + j * NSUB) < NCHG
            vs = jnp.full((L,), jnp.where(valid, 1.0, 0.0), jnp.float32)
            mxl = xlb[b]
            mxr = xrb[b]
            mee = eeb[b]

            @plsc.parallel_loop(0, CHUNK, unroll=4)
            def _alpha(e):
                acc = zv
                for jj in range(C // L):
                    sl = pl.ds(jj * L, L)
                    m = mxl[e, sl] + mxr[e, sl] + mee[e, sl]
                    m = jnp.maximum(m, NEG * m)
                    acc = acc + m * attv[sl]
                ex = jnp.exp(jnp.full((L,), jnp.sum(acc), jnp.float32)) * vs
                plsc.store_scatter(exb[b], [jnp.full((L,), e, jnp.int32)],
                                   ex, mask=lane0)

            @plsc.parallel_loop(0, CHUNK, unroll=4)
            def _scale(e):
                exv = plsc.load_gather(exb[b],
                                       [jnp.full((L,), e, jnp.int32)])
                for jj in range(C // L):
                    sl = pl.ds(jj * L, L)
                    mxl[e, sl] = mxl[e, sl] * exv

            for g in range(NGRP):
                sl = pl.ds(g * L, L)
                eib[b][1, sl] = eib[b][1, sl] - hoff
            pltpu.async_copy(exb[b], denf_sh.at[eib[b].at[1]], semsd[b],
                             add=True)
            pltpu.async_copy(mxl, out_sh.at[eib[b].at[1]], semso[b],
                             add=True)

        _stage(0, 0)

        def _pair(jo, _):
            for b in range(2):
                j = jo * 2 + b

                @pl.when(j < JPAD)
                def _():
                    k = tid + j * NSUB
                    base = jnp.minimum(k, NCHG - 1) * CHUNK
                    pltpu.make_async_copy(xl.at[eib[b].at[0]], xlb[b],
                                          semxl[b]).wait()
                    pltpu.make_async_copy(xr.at[eib[b].at[1]], xrb[b],
                                          semxr[b]).wait()
                    pltpu.make_async_copy(
                        ee.at[pl.ds(c * E + base, CHUNK)], eeb[b],
                        semee[b]).wait()

                    @pl.when(j + 1 < JPAD)
                    def _():
                        @pl.when(j >= 1)
                        def _():
                            pltpu.make_async_copy(
                                exb[1 - b], denf_sh.at[eib[1 - b].at[1]],
                                semsd[1 - b]).wait()
                            pltpu.make_async_copy(
                                xlb[1 - b], out_sh.at[eib[1 - b].at[1]],
                                semso[1 - b]).wait()
                        _stage(j + 1, 1 - b)
                    _compute(j, b)
            return 0
        lax.fori_loop(0, (JPAD + 1) // 2, _pair, 0)
        for m in (JPAD - 2, JPAD - 1):
            bm = m % 2
            pltpu.make_async_copy(exb[bm], denf_sh.at[eib[bm].at[1]],
                                  semsd[bm]).wait()
            pltpu.make_async_copy(xlb[bm], out_sh.at[eib[bm].at[1]],
                                  semso[bm]).wait()

        plsc.subcore_barrier()

        def _wb(rows, rowbase):
            pltpu.sync_copy(out_sh.at[pl.ds(rowbase, rows)],
                            xrb0.at[pl.ds(0, rows)])
            pltpu.sync_copy(denf_sh.at[pl.ds(rowbase, rows)],
                            exb0.at[pl.ds(0, rows)])

            @plsc.parallel_loop(0, rows, unroll=4)
            def _nrm(r):
                dv = plsc.load_gather(exb0,
                                      [jnp.full((L,), r, jnp.int32)])
                rcv = 1.0 / (dv + 1e-16)
                for jj in range(C // L):
                    sl = pl.ds(jj * L, L)
                    v = xrb0[r, sl] * rcv + biasv[sl]
                    xrb0[r, sl] = jnp.maximum(v, 0.0)
            pltpu.sync_copy(
                xrb0.at[pl.ds(0, rows)],
                outref.at[pl.ds(rowbase, rows), pl.ds(c * C, C)])

        def _wchunk(jw, _):
            k = tid + jw * NSUB

            @pl.when(k < WFULL)
            def _():
                _wb(CHUNK, k * CHUNK)

            @pl.when(k == WFULL)
            def _():
                _wb(WTAIL, WFULL * CHUNK)
            return 0
        lax.fori_loop(0, (WFULL + NSUB) // NSUB, _wchunk, 0)
        plsc.subcore_barrier()


_sc_call = pl.kernel(
    _sc_body,
    out_type=(
        jax.ShapeDtypeStruct((N, H * C), jnp.float32),
        jax.ShapeDtypeStruct((N, H * C), jnp.float32),
    ),
    mesh=plsc.VectorSubcoreMesh(core_axis_name="c", subcore_axis_name="s"),
    compiler_params=pltpu.CompilerParams(needs_layout_passes=False),
    scratch_types=[
        pltpu.VMEM((CHUNK, C), jnp.float32),
        pltpu.VMEM((CHUNK, C), jnp.float32),
        pltpu.VMEM((CHUNK, C), jnp.float32),
        pltpu.VMEM((CHUNK, C), jnp.float32),
        pltpu.VMEM((CHUNK, C), jnp.float32),
        pltpu.VMEM((CHUNK, C), jnp.float32),
        pltpu.VMEM((2, CHUNK), jnp.int32),
        pltpu.VMEM((2, CHUNK), jnp.int32),
        pltpu.VMEM((CHUNK,), jnp.float32),
        pltpu.VMEM((CHUNK,), jnp.float32),
        pltpu.VMEM((C,), jnp.float32),
        pltpu.VMEM((C,), jnp.float32),
        pltpu.VMEM_SHARED((N, C), jnp.float32),
        pltpu.VMEM_SHARED((NPAD,), jnp.float32),
        pltpu.SemaphoreType.DMA,
        pltpu.SemaphoreType.DMA,
        pltpu.SemaphoreType.DMA,
        pltpu.SemaphoreType.DMA,
        pltpu.SemaphoreType.DMA,
        pltpu.SemaphoreType.DMA,
        pltpu.SemaphoreType.DMA,
        pltpu.SemaphoreType.DMA,
        pltpu.SemaphoreType.DMA,
        pltpu.SemaphoreType.DMA,
    ],
)


@jax.jit
def kernel(s, t, edges, edge_weight,
           sWl, sWr, sWe, satt, sbias,
           tWl, tWr, tWe, tatt, tbias):
    pa = _project(s, jnp.concatenate([sWl, tWr], axis=1), 1000)
    pb = _project(t, jnp.concatenate([sWr, tWl], axis=1), 1000)
    pc = _project(edge_weight, jnp.concatenate([sWe, tWe], axis=1), 2000)
    xls = pa[0:2].reshape(2 * N, C)
    xrt = pa[2:4].reshape(2 * N, C)
    xrs = pb[0:2].reshape(2 * N, C)
    xlt = pb[2:4].reshape(2 * N, C)
    ees = pc[0:2].reshape(2 * E, C)
    eet = pc[2:4].reshape(2 * E, C)

    att4 = jnp.concatenate([satt, tatt], axis=0)
    attb = att4.reshape(4 * C)
    bias4 = jnp.concatenate(
        [sbias.reshape(H, C), tbias.reshape(H, C)], axis=0)
    biasb = bias4.reshape(4 * C)

    outs, outt = _sc_call(edges[0], edges[1], xls, xrs, ees, xlt, xrt,
                          eet, attb, biasb)
    return (outs, outt, edges, edge_weight)


# speedup vs baseline: 27.0069x; 1.1346x over previous
import functools

import jax
import jax.numpy as jnp
from jax import lax
from jax.experimental import pallas as pl
from jax.experimental.pallas import tpu as pltpu
from jax.experimental.pallas import tpu_sc as plsc

N = 10000
E = 160000
D = 256
H = 2
C = 128
NEG = 0.2
L = 16
NSUB = 16
CHUNK = 64
NGRP = CHUNK // L
NCHG = E // CHUNK
JPAD = (NCHG + NSUB - 1) // NSUB
NPAD = JPAD * CHUNK
WFULL = N // CHUNK
WTAIL = N - WFULL * CHUNK


def _mm_body(x_ref, w_ref, o_ref):
    o_ref[0] = jnp.dot(x_ref[...], w_ref[...],
                       preferred_element_type=jnp.float32)


def _project(x, w, bn):
    M, K = x.shape
    G = w.shape[1] // 128
    return pl.pallas_call(
        _mm_body,
        grid=(G, M // bn),
        in_specs=[
            pl.BlockSpec((bn, K), lambda g, i: (i, 0)),
            pl.BlockSpec((K, 128), lambda g, i: (0, g)),
        ],
        out_specs=pl.BlockSpec((1, bn, 128), lambda g, i: (g, i, 0)),
        out_shape=jax.ShapeDtypeStruct((G, M, 128), jnp.float32),
    )(x, w)


def _sc_body(e0, e1, xls, xrs, ees, xlt, xrt, eet, attb, biasb,
             outs, outt,
             xlb0, xlb1, xrb0, xrb1, eeb,
             eib0, eib1, eib2, eib3,
             exb0, exb1, attv, biasv,
             out_sh, denf_sh,
             semxl0, semxl1, semxr0, semxr1, semee,
             semso0, semso1, semsd0, semsd1,
             semei0, semei1, semei2, semei3):
    c = lax.axis_index("c")
    tid = lax.axis_index("s")
    iota = lax.iota(jnp.int32, L)
    zv = jnp.zeros((L,), jnp.float32)
    lane0 = iota == 0
    xlb = (xlb0, xlb1)
    xrb = (xrb0, xrb1)
    eib = (eib0, eib1, eib2, eib3)
    semei = (semei0, semei1, semei2, semei3)
    semxl = (semxl0, semxl1)
    semxr = (semxr0, semxr1)
    semso = (semso0, semso1)
    semsd = (semsd0, semsd1)
    exb = (exb0, exb1)

    for d, (xl, xr, ee, esrc, edst, outref) in enumerate((
            (xls, xrs, ees, e0, e1, outs),
            (xlt, xrt, eet, e1, e0, outt))):
        q = d * 2 + c
        pltpu.sync_copy(attb.at[pl.ds(q * C, C)], attv)
        pltpu.sync_copy(biasb.at[pl.ds(q * C, C)], biasv)
        hoff = c * N

        def _zrow(r, _):
            for jz in range(C // L):
                xlb0[r, pl.ds(jz * L, L)] = zv
            return 0
        lax.fori_loop(0, CHUNK, _zrow, 0)
        for g in range(NGRP):
            exb0[pl.ds(g * L, L)] = zv

        def _zout(jz, _):
            k = tid + jz * NSUB

            @pl.when(k < WFULL)
            def _():
                pltpu.sync_copy(xlb0, out_sh.at[pl.ds(k * CHUNK, CHUNK)])

            @pl.when(k == WFULL)
            def _():
                pltpu.sync_copy(xlb0.at[pl.ds(0, WTAIL)],
                                out_sh.at[pl.ds(WFULL * CHUNK, WTAIL)])

            @pl.when(k < JPAD)
            def _():
                pltpu.sync_copy(exb0, denf_sh.at[pl.ds(k * CHUNK, CHUNK)])
            return 0
        lax.fori_loop(0, (JPAD + NSUB - 1) // NSUB, _zout, 0)

        plsc.subcore_barrier()

        def _ifetch(j, q):
            k = tid + j * NSUB
            base = jnp.minimum(k, NCHG - 1) * CHUNK
            pltpu.async_copy(esrc.at[pl.ds(base, CHUNK)], eib[q].at[0],
                             semei[q])
            pltpu.async_copy(edst.at[pl.ds(base, CHUNK)], eib[q].at[1],
                             semei[q])

        def _gissue(j, q, b):
            k = tid + j * NSUB
            base = jnp.minimum(k, NCHG - 1) * CHUNK
            pltpu.make_async_copy(esrc.at[pl.ds(base, CHUNK)],
                                  eib[q].at[0], semei[q]).wait()
            pltpu.make_async_copy(edst.at[pl.ds(base, CHUNK)],
                                  eib[q].at[1], semei[q]).wait()
            for g in range(NGRP):
                sl = pl.ds(g * L, L)
                eib[q][0, sl] = eib[q][0, sl] + hoff
                eib[q][1, sl] = eib[q][1, sl] + hoff
            pltpu.async_copy(xl.at[eib[q].at[0]], xlb[b], semxl[b])
            pltpu.async_copy(xr.at[eib[q].at[1]], xrb[b], semxr[b])

        def _compute(j, q, b):
            valid = (tid + j * NSUB) < NCHG
            vs = jnp.full((L,), jnp.where(valid, 1.0, 0.0), jnp.float32)
            mxl = xlb[b]
            mxr = xrb[b]
            mee = eeb

            @plsc.parallel_loop(0, CHUNK, unroll=4)
            def _alpha(e):
                acc = zv
                for jj in range(C // L):
                    sl = pl.ds(jj * L, L)
                    m = mxl[e, sl] + mxr[e, sl] + mee[e, sl]
                    m = jnp.maximum(m, NEG * m)
                    acc = acc + m * attv[sl]
                ex = jnp.exp(jnp.full((L,), jnp.sum(acc), jnp.float32)) * vs
                plsc.store_scatter(exb[b], [jnp.full((L,), e, jnp.int32)],
                                   ex, mask=lane0)

            @pl.when(j + 1 < JPAD)
            def _():
                k1 = tid + (j + 1) * NSUB
                base1 = jnp.minimum(k1, NCHG - 1) * CHUNK
                pltpu.async_copy(ee.at[pl.ds(c * E + base1, CHUNK)], eeb,
                                 semee)

            @plsc.parallel_loop(0, CHUNK, unroll=4)
            def _scale(e):
                exv = plsc.load_gather(exb[b],
                                       [jnp.full((L,), e, jnp.int32)])
                for jj in range(C // L):
                    sl = pl.ds(jj * L, L)
                    mxl[e, sl] = mxl[e, sl] * exv

            for g in range(NGRP):
                sl = pl.ds(g * L, L)
                eib[q][1, sl] = eib[q][1, sl] - hoff
            pltpu.async_copy(exb[b], denf_sh.at[eib[q].at[1]], semsd[b],
                             add=True)
            pltpu.async_copy(mxl, out_sh.at[eib[q].at[1]], semso[b],
                             add=True)

        _ifetch(0, 0)
        _ifetch(1, 1)
        _ifetch(2, 2)
        _gissue(0, 0, 0)
        base00 = jnp.minimum(tid, NCHG - 1) * CHUNK
        pltpu.async_copy(ee.at[pl.ds(c * E + base00, CHUNK)], eeb, semee)

        def _quad(jo, _):
            for b4 in range(4):
                j = jo * 4 + b4
                b = b4 % 2

                @pl.when(j < JPAD)
                def _():
                    k = tid + j * NSUB
                    base = jnp.minimum(k, NCHG - 1) * CHUNK
                    pltpu.make_async_copy(xl.at[eib[b4].at[0]], xlb[b],
                                          semxl[b]).wait()
                    pltpu.make_async_copy(xr.at[eib[b4].at[1]], xrb[b],
                                          semxr[b]).wait()
                    pltpu.make_async_copy(
                        ee.at[pl.ds(c * E + base, CHUNK)], eeb,
                        semee).wait()

                    @pl.when(j + 3 < JPAD)
                    def _():
                        _ifetch(j + 3, (b4 + 3) % 4)

                    @pl.when(j + 1 < JPAD)
                    def _():
                        @pl.when(j >= 1)
                        def _():
                            pltpu.make_async_copy(
                                exb[1 - b],
                                denf_sh.at[eib[(b4 + 3) % 4].at[1]],
                                semsd[1 - b]).wait()
                            pltpu.make_async_copy(
                                xlb[1 - b],
                                out_sh.at[eib[(b4 + 3) % 4].at[1]],
                                semso[1 - b]).wait()
                        _gissue(j + 1, (b4 + 1) % 4, 1 - b)
                    _compute(j, b4, b)
            return 0
        lax.fori_loop(0, (JPAD + 3) // 4, _quad, 0)
        for m in (JPAD - 2, JPAD - 1):
            bm = m % 2
            qm = m % 4
            pltpu.make_async_copy(exb[bm], denf_sh.at[eib[qm].at[1]],
                                  semsd[bm]).wait()
            pltpu.make_async_copy(xlb[bm], out_sh.at[eib[qm].at[1]],
                                  semso[bm]).wait()

        plsc.subcore_barrier()

        def _wb(rows, rowbase):
            pltpu.sync_copy(out_sh.at[pl.ds(rowbase, rows)],
                            xrb0.at[pl.ds(0, rows)])
            pltpu.sync_copy(denf_sh.at[pl.ds(rowbase, rows)],
                            exb0.at[pl.ds(0, rows)])

            @plsc.parallel_loop(0, rows, unroll=4)
            def _nrm(r):
                dv = plsc.load_gather(exb0,
                                      [jnp.full((L,), r, jnp.int32)])
                rcv = 1.0 / (dv + 1e-16)
                for jj in range(C // L):
                    sl = pl.ds(jj * L, L)
                    v = xrb0[r, sl] * rcv + biasv[sl]
                    xrb0[r, sl] = jnp.maximum(v, 0.0)
            pltpu.sync_copy(
                xrb0.at[pl.ds(0, rows)],
                outref.at[pl.ds(rowbase, rows), pl.ds(c * C, C)])

        def _wchunk(jw, _):
            k = tid + jw * NSUB

            @pl.when(k < WFULL)
            def _():
                _wb(CHUNK, k * CHUNK)

            @pl.when(k == WFULL)
            def _():
                _wb(WTAIL, WFULL * CHUNK)
            return 0
        lax.fori_loop(0, (WFULL + NSUB) // NSUB, _wchunk, 0)
        plsc.subcore_barrier()


_sc_call = pl.kernel(
    _sc_body,
    out_type=(
        jax.ShapeDtypeStruct((N, H * C), jnp.float32),
        jax.ShapeDtypeStruct((N, H * C), jnp.float32),
    ),
    mesh=plsc.VectorSubcoreMesh(core_axis_name="c", subcore_axis_name="s"),
    compiler_params=pltpu.CompilerParams(needs_layout_passes=False),
    scratch_types=[
        pltpu.VMEM((CHUNK, C), jnp.float32),
        pltpu.VMEM((CHUNK, C), jnp.float32),
        pltpu.VMEM((CHUNK, C), jnp.float32),
        pltpu.VMEM((CHUNK, C), jnp.float32),
        pltpu.VMEM((CHUNK, C), jnp.float32),
        pltpu.VMEM((2, CHUNK), jnp.int32),
        pltpu.VMEM((2, CHUNK), jnp.int32),
        pltpu.VMEM((2, CHUNK), jnp.int32),
        pltpu.VMEM((2, CHUNK), jnp.int32),
        pltpu.VMEM((CHUNK,), jnp.float32),
        pltpu.VMEM((CHUNK,), jnp.float32),
        pltpu.VMEM((C,), jnp.float32),
        pltpu.VMEM((C,), jnp.float32),
        pltpu.VMEM_SHARED((N, C), jnp.float32),
        pltpu.VMEM_SHARED((NPAD,), jnp.float32),
        pltpu.SemaphoreType.DMA,
        pltpu.SemaphoreType.DMA,
        pltpu.SemaphoreType.DMA,
        pltpu.SemaphoreType.DMA,
        pltpu.SemaphoreType.DMA,
        pltpu.SemaphoreType.DMA,
        pltpu.SemaphoreType.DMA,
        pltpu.SemaphoreType.DMA,
        pltpu.SemaphoreType.DMA,
        pltpu.SemaphoreType.DMA,
        pltpu.SemaphoreType.DMA,
        pltpu.SemaphoreType.DMA,
        pltpu.SemaphoreType.DMA,
    ],
)


@jax.jit
def kernel(s, t, edges, edge_weight,
           sWl, sWr, sWe, satt, sbias,
           tWl, tWr, tWe, tatt, tbias):
    pa = _project(s, jnp.concatenate([sWl, tWr], axis=1), 1000)
    pb = _project(t, jnp.concatenate([sWr, tWl], axis=1), 1000)
    pc = _project(edge_weight, jnp.concatenate([sWe, tWe], axis=1), 2000)
    xls = pa[0:2].reshape(2 * N, C)
    xrt = pa[2:4].reshape(2 * N, C)
    xrs = pb[0:2].reshape(2 * N, C)
    xlt = pb[2:4].reshape(2 * N, C)
    ees = pc[0:2].reshape(2 * E, C)
    eet = pc[2:4].reshape(2 * E, C)

    att4 = jnp.concatenate([satt, tatt], axis=0)
    attb = att4.reshape(4 * C)
    bias4 = jnp.concatenate(
        [sbias.reshape(H, C), tbias.reshape(H, C)], axis=0)
    biasb = bias4.reshape(4 * C)

    outs, outt = _sc_call(edges[0], edges[1], xls, xrs, ees, xlt, xrt,
                          eet, attb, biasb)
    return (outs, outt, edges, edge_weight)


# speedup vs baseline: 38.5462x; 1.4273x over previous
import functools

import jax
import jax.numpy as jnp
from jax import lax
from jax.experimental import pallas as pl
from jax.experimental.pallas import tpu as pltpu
from jax.experimental.pallas import tpu_sc as plsc

N = 10000
E = 160000
D = 256
H = 2
C = 128
NEG = 0.2
L = 16
NSUB = 16
CHUNK = 64
NGRP = CHUNK // L
NCHG = E // CHUNK
JPAD = (NCHG + NSUB - 1) // NSUB
NPAD = JPAD * CHUNK
WFULL = N // CHUNK
WTAIL = N - WFULL * CHUNK


def _mm_body(x_ref, w_ref, o_ref):
    o_ref[0] = jnp.dot(x_ref[...], w_ref[...],
                       preferred_element_type=jnp.float32)


def _project_w(x, w, bn):
    M, K = x.shape
    W = w.shape[1]
    return pl.pallas_call(
        lambda x_ref, w_ref, o_ref: o_ref.__setitem__(
            ..., jnp.dot(x_ref[...], w_ref[...],
                         preferred_element_type=jnp.float32)),
        grid=(M // bn,),
        in_specs=[
            pl.BlockSpec((bn, K), lambda i: (i, 0)),
            pl.BlockSpec((K, W), lambda i: (0, 0)),
        ],
        out_specs=pl.BlockSpec((bn, W), lambda i: (i, 0)),
        out_shape=jax.ShapeDtypeStruct((M, W), jnp.float32),
    )(x, w)


def _project(x, w, bn):
    M, K = x.shape
    G = w.shape[1] // 128
    return pl.pallas_call(
        _mm_body,
        grid=(G, M // bn),
        in_specs=[
            pl.BlockSpec((bn, K), lambda g, i: (i, 0)),
            pl.BlockSpec((K, 128), lambda g, i: (0, g)),
        ],
        out_specs=pl.BlockSpec((1, bn, 128), lambda g, i: (g, i, 0)),
        out_shape=jax.ShapeDtypeStruct((G, M, 128), jnp.float32),
    )(x, w)


def _sc_body(e0, e1, paf, pbf, eef, attb, biasb,
             outs, outt,
             xlb0, xlb1, xrb0, xrb1, eeb,
             eib0, eib1, eib2, eib3,
             exb0, exb1, attv, biasv,
             out_sh, denf_sh,
             semxl0, semxl1, semxr0, semxr1, semee,
             semso0, semso1, semsd0, semsd1,
             semei0, semei1, semei2, semei3):
    c = lax.axis_index("c")
    tid = lax.axis_index("s")
    iota = lax.iota(jnp.int32, L)
    zv = jnp.zeros((L,), jnp.float32)
    lane0 = iota == 0
    xlb = (xlb0, xlb1)
    xrb = (xrb0, xrb1)
    eib = (eib0, eib1, eib2, eib3)
    semei = (semei0, semei1, semei2, semei3)
    semxl = (semxl0, semxl1)
    semxr = (semxr0, semxr1)
    semso = (semso0, semso1)
    semsd = (semsd0, semsd1)
    exb = (exb0, exb1)

    for d, (xl, xr, esrc, edst, outref) in enumerate((
            (paf, pbf, e0, e1, outs),
            (pbf, paf, e1, e0, outt))):
        q = d * 2 + c
        pltpu.sync_copy(attb.at[pl.ds(q * C, C)], attv)
        pltpu.sync_copy(biasb.at[pl.ds(q * C, C)], biasv)
        hoff = q * N

        def _zrow(r, _):
            for jz in range(C // L):
                xlb0[r, pl.ds(jz * L, L)] = zv
            return 0
        lax.fori_loop(0, CHUNK, _zrow, 0)
        for g in range(NGRP):
            exb0[pl.ds(g * L, L)] = zv

        def _zout(jz, _):
            k = tid + jz * NSUB

            @pl.when(k < WFULL)
            def _():
                pltpu.sync_copy(xlb0, out_sh.at[pl.ds(k * CHUNK, CHUNK)])

            @pl.when(k == WFULL)
            def _():
                pltpu.sync_copy(xlb0.at[pl.ds(0, WTAIL)],
                                out_sh.at[pl.ds(WFULL * CHUNK, WTAIL)])

            @pl.when(k < JPAD)
            def _():
                pltpu.sync_copy(exb0, denf_sh.at[pl.ds(k * CHUNK, CHUNK)])
            return 0
        lax.fori_loop(0, (JPAD + NSUB - 1) // NSUB, _zout, 0)

        plsc.subcore_barrier()

        def _ifetch(j, q):
            k = tid + j * NSUB
            base = jnp.minimum(k, NCHG - 1) * CHUNK
            pltpu.async_copy(esrc.at[pl.ds(base, CHUNK)], eib[q].at[0],
                             semei[q])
            pltpu.async_copy(edst.at[pl.ds(base, CHUNK)], eib[q].at[1],
                             semei[q])

        def _gissue(j, q, b):
            k = tid + j * NSUB
            base = jnp.minimum(k, NCHG - 1) * CHUNK
            pltpu.make_async_copy(esrc.at[pl.ds(base, CHUNK)],
                                  eib[q].at[0], semei[q]).wait()
            pltpu.make_async_copy(edst.at[pl.ds(base, CHUNK)],
                                  eib[q].at[1], semei[q]).wait()
            for g in range(NGRP):
                sl = pl.ds(g * L, L)
                eib[q][0, sl] = eib[q][0, sl] + hoff
                eib[q][1, sl] = eib[q][1, sl] + hoff
            pltpu.async_copy(xl.at[eib[q].at[0]], xlb[b], semxl[b])
            pltpu.async_copy(xr.at[eib[q].at[1]], xrb[b], semxr[b])

        def _compute(j, q, b):
            valid = (tid + j * NSUB) < NCHG
            vs = jnp.full((L,), jnp.where(valid, 1.0, 0.0), jnp.float32)
            mxl = xlb[b]
            mxr = xrb[b]
            mee = eeb

            @plsc.parallel_loop(0, CHUNK, unroll=4)
            def _alpha(e):
                acc = zv
                for jj in range(C // L):
                    sl = pl.ds(jj * L, L)
                    m = mxl[e, sl] + mxr[e, sl] + mee[e, sl]
                    m = jnp.maximum(m, NEG * m)
                    acc = acc + m * attv[sl]
                ex = jnp.exp(jnp.full((L,), jnp.sum(acc), jnp.float32)) * vs
                plsc.store_scatter(exb[b], [jnp.full((L,), e, jnp.int32)],
                                   ex, mask=lane0)

            @pl.when(j + 1 < JPAD)
            def _():
                k1 = tid + (j + 1) * NSUB
                base1 = jnp.minimum(k1, NCHG - 1) * CHUNK
                pltpu.async_copy(
                    eef.at[pl.ds(base1, CHUNK), pl.ds(q * C, C)], eeb,
                    semee)

            @plsc.parallel_loop(0, CHUNK, unroll=4)
            def _scale(e):
                exv = plsc.load_gather(exb[b],
                                       [jnp.full((L,), e, jnp.int32)])
                for jj in range(C // L):
                    sl = pl.ds(jj * L, L)
                    mxl[e, sl] = mxl[e, sl] * exv

            for g in range(NGRP):
                sl = pl.ds(g * L, L)
                eib[q][1, sl] = eib[q][1, sl] - hoff
            pltpu.async_copy(exb[b], denf_sh.at[eib[q].at[1]], semsd[b],
                             add=True)
            pltpu.async_copy(mxl, out_sh.at[eib[q].at[1]], semso[b],
                             add=True)

        _ifetch(0, 0)
        _ifetch(1, 1)
        _ifetch(2, 2)
        _gissue(0, 0, 0)
        base00 = jnp.minimum(tid, NCHG - 1) * CHUNK
        pltpu.async_copy(eef.at[pl.ds(base00, CHUNK), pl.ds(q * C, C)],
                         eeb, semee)

        def _quad(jo, _):
            for b4 in range(4):
                j = jo * 4 + b4
                b = b4 % 2

                @pl.when(j < JPAD)
                def _():
                    k = tid + j * NSUB
                    base = jnp.minimum(k, NCHG - 1) * CHUNK
                    pltpu.make_async_copy(xl.at[eib[b4].at[0]], xlb[b],
                                          semxl[b]).wait()
                    pltpu.make_async_copy(xr.at[eib[b4].at[1]], xrb[b],
                                          semxr[b]).wait()
                    pltpu.make_async_copy(
                        eef.at[pl.ds(base, CHUNK), pl.ds(q * C, C)], eeb,
                        semee).wait()

                    @pl.when(j + 3 < JPAD)
                    def _():
                        _ifetch(j + 3, (b4 + 3) % 4)

                    @pl.when(j + 1 < JPAD)
                    def _():
                        @pl.when(j >= 1)
                        def _():
                            pltpu.make_async_copy(
                                exb[1 - b],
                                denf_sh.at[eib[(b4 + 3) % 4].at[1]],
                                semsd[1 - b]).wait()
                            pltpu.make_async_copy(
                                xlb[1 - b],
                                out_sh.at[eib[(b4 + 3) % 4].at[1]],
                                semso[1 - b]).wait()
                        _gissue(j + 1, (b4 + 1) % 4, 1 - b)
                    _compute(j, b4, b)
            return 0
        lax.fori_loop(0, (JPAD + 3) // 4, _quad, 0)
        for m in (JPAD - 2, JPAD - 1):
            bm = m % 2
            qm = m % 4
            pltpu.make_async_copy(exb[bm], denf_sh.at[eib[qm].at[1]],
                                  semsd[bm]).wait()
            pltpu.make_async_copy(xlb[bm], out_sh.at[eib[qm].at[1]],
                                  semso[bm]).wait()

        plsc.subcore_barrier()

        def _wb(rows, rowbase):
            pltpu.sync_copy(out_sh.at[pl.ds(rowbase, rows)],
                            xrb0.at[pl.ds(0, rows)])
            pltpu.sync_copy(denf_sh.at[pl.ds(rowbase, rows)],
                            exb0.at[pl.ds(0, rows)])

            @plsc.parallel_loop(0, rows, unroll=4)
            def _nrm(r):
                dv = plsc.load_gather(exb0,
                                      [jnp.full((L,), r, jnp.int32)])
                rcv = 1.0 / (dv + 1e-16)
                for jj in range(C // L):
                    sl = pl.ds(jj * L, L)
                    v = xrb0[r, sl] * rcv + biasv[sl]
                    xrb0[r, sl] = jnp.maximum(v, 0.0)
            pltpu.sync_copy(
                xrb0.at[pl.ds(0, rows)],
                outref.at[pl.ds(rowbase, rows), pl.ds(c * C, C)])

        def _wchunk(jw, _):
            k = tid + jw * NSUB

            @pl.when(k < WFULL)
            def _():
                _wb(CHUNK, k * CHUNK)

            @pl.when(k == WFULL)
            def _():
                _wb(WTAIL, WFULL * CHUNK)
            return 0
        lax.fori_loop(0, (WFULL + NSUB) // NSUB, _wchunk, 0)
        plsc.subcore_barrier()


_sc_call = pl.kernel(
    _sc_body,
    out_type=(
        jax.ShapeDtypeStruct((N, H * C), jnp.float32),
        jax.ShapeDtypeStruct((N, H * C), jnp.float32),
    ),
    mesh=plsc.VectorSubcoreMesh(core_axis_name="c", subcore_axis_name="s"),
    compiler_params=pltpu.CompilerParams(needs_layout_passes=False),
    scratch_types=[
        pltpu.VMEM((CHUNK, C), jnp.float32),
        pltpu.VMEM((CHUNK, C), jnp.float32),
        pltpu.VMEM((CHUNK, C), jnp.float32),
        pltpu.VMEM((CHUNK, C), jnp.float32),
        pltpu.VMEM((CHUNK, C), jnp.float32),
        pltpu.VMEM((2, CHUNK), jnp.int32),
        pltpu.VMEM((2, CHUNK), jnp.int32),
        pltpu.VMEM((2, CHUNK), jnp.int32),
        pltpu.VMEM((2, CHUNK), jnp.int32),
        pltpu.VMEM((CHUNK,), jnp.float32),
        pltpu.VMEM((CHUNK,), jnp.float32),
        pltpu.VMEM((C,), jnp.float32),
        pltpu.VMEM((C,), jnp.float32),
        pltpu.VMEM_SHARED((N, C), jnp.float32),
        pltpu.VMEM_SHARED((NPAD,), jnp.float32),
        pltpu.SemaphoreType.DMA,
        pltpu.SemaphoreType.DMA,
        pltpu.SemaphoreType.DMA,
        pltpu.SemaphoreType.DMA,
        pltpu.SemaphoreType.DMA,
        pltpu.SemaphoreType.DMA,
        pltpu.SemaphoreType.DMA,
        pltpu.SemaphoreType.DMA,
        pltpu.SemaphoreType.DMA,
        pltpu.SemaphoreType.DMA,
        pltpu.SemaphoreType.DMA,
        pltpu.SemaphoreType.DMA,
        pltpu.SemaphoreType.DMA,
    ],
)


@jax.jit
def kernel(s, t, edges, edge_weight,
           sWl, sWr, sWe, satt, sbias,
           tWl, tWr, tWe, tatt, tbias):
    pa = _project(s, jnp.concatenate([sWl, tWr], axis=1), 1000)
    pb = _project(t, jnp.concatenate([sWr, tWl], axis=1), 1000)
    paf = pa.reshape(4 * N, C)
    pbf = pb.reshape(4 * N, C)
    eef = _project_w(edge_weight, jnp.concatenate([sWe, tWe], axis=1), 2000)

    att4 = jnp.concatenate([satt, tatt], axis=0)
    attb = att4.reshape(4 * C)
    bias4 = jnp.concatenate(
        [sbias.reshape(H, C), tbias.reshape(H, C)], axis=0)
    biasb = bias4.reshape(4 * C)

    outs, outt = _sc_call(edges[0], edges[1], paf, pbf, eef, attb, biasb)
    return (outs, outt, edges, edge_weight)
